# baseline (speedup 1.0000x reference)
import functools

import jax
import jax.numpy as jnp
from jax import lax
from jax.experimental import pallas as pl
from jax.experimental.pallas import tpu as pltpu
from jax.experimental.pallas import tpu_sc as plsc

N = 10000
E = 320000
D = 128
C = 16
NC = 2
NS = 16
NW = NC * NS
EPW = E // NW
K = 128
EPW_P = 10240
CH = EPW_P // K
N_PAD = 10112
STR = N_PAD // NS
N_PADD = 12288
STRD = N_PADD // NS
DEGW = 16
SHIFT = 14
MASK = (1 << SHIFT) - 1
R = 1000


def _sc_mesh():
    return plsc.VectorSubcoreMesh(core_axis_name="c", subcore_axis_name="s")


def _unpack_dst(pk_v, j, u, row):
    for i in range(K // 16):
        v = pk_v[j, pl.ds(i * 16, 16)]
        u[row, pl.ds(i * 16, 16)] = lax.shift_right_logical(v, SHIFT)


def _unpack_both(pk_v, j, u):
    for i in range(K // 16):
        v = pk_v[j, pl.ds(i * 16, 16)]
        u[0, pl.ds(i * 16, 16)] = v & MASK
        u[1, pl.ds(i * 16, 16)] = lax.shift_right_logical(v, SHIFT)


def _sc_degree(pk3, ones_k, zeros_k):
    @functools.partial(
        pl.kernel,
        out_type=jax.ShapeDtypeStruct((NC, N_PADD), jnp.float32),
        mesh=_sc_mesh(),
        scratch_types=[
            pltpu.VMEM((CH, K), jnp.int32),
            pltpu.VMEM((2, K), jnp.int32),
            pltpu.VMEM((K,), jnp.float32),
            pltpu.VMEM_SHARED((N_PADD,), jnp.float32),
        ],
    )
    def k(pk_hbm, ones_hbm, zeros_hbm, out_hbm, pk_v, u, ones_v, acc):
        cid = lax.axis_index("c")
        sid = lax.axis_index("s")
        wid = sid * NC + cid
        s = sid * STRD
        pltpu.sync_copy(pk_hbm.at[wid], pk_v)
        pltpu.sync_copy(ones_hbm, ones_v)
        pltpu.sync_copy(zeros_hbm, acc.at[pl.ds(s, STRD)])
        plsc.subcore_barrier()

        def body(j, carry):
            _unpack_dst(pk_v, j, u, 0)
            pltpu.sync_copy(ones_v, acc.at[u.at[0]], add=True)
            return carry

        lax.fori_loop(0, CH, body, 0)
        plsc.subcore_barrier()
        pltpu.sync_copy(acc.at[pl.ds(s, STRD)], out_hbm.at[cid].at[pl.ds(s, STRD)])

    return k(pk3, ones_k, zeros_k)


def _sc_scatter_rows(table, pk3, zeros_row):
    @functools.partial(
        pl.kernel,
        out_type=jax.ShapeDtypeStruct((NC, N_PAD, D), jnp.float32),
        mesh=_sc_mesh(),
        scratch_types=[
            pltpu.VMEM((CH, K), jnp.int32),
            pltpu.VMEM((2, K), jnp.int32),
            pltpu.VMEM((2, K), jnp.int32),
            pltpu.VMEM((K, D), jnp.float32),
            pltpu.VMEM((K, D), jnp.float32),
            pltpu.VMEM_SHARED((N_PAD, D), jnp.float32),
            pltpu.SemaphoreType.DMA,
            pltpu.SemaphoreType.DMA,
            pltpu.SemaphoreType.DMA,
            pltpu.SemaphoreType.DMA,
        ],
    )
    def k(table_hbm, pk_hbm, z_hbm, out_hbm,
          pk_v, ua, ub, ra, rb, acc, sa, sb, ssa, ssb):
        cid = lax.axis_index("c")
        sid = lax.axis_index("s")
        wid = sid * NC + cid
        s = sid * STR
        pltpu.sync_copy(pk_hbm.at[wid], pk_v)
        pltpu.sync_copy(z_hbm, acc.at[pl.ds(s, STR)])
        plsc.subcore_barrier()

        _unpack_both(pk_v, 0, ua)
        pltpu.async_copy(table_hbm.at[ua.at[0]], ra, sa)
        _unpack_both(pk_v, 1, ub)
        pltpu.async_copy(table_hbm.at[ub.at[0]], rb, sb)

        def body(jj, carry):
            j0 = jj * 2
            pltpu.make_async_copy(table_hbm.at[ua.at[0]], ra, sa).wait()
            pltpu.async_copy(ra, acc.at[ua.at[1]], ssa, add=True)
            pltpu.make_async_copy(table_hbm.at[ub.at[0]], rb, sb).wait()
            pltpu.async_copy(rb, acc.at[ub.at[1]], ssb, add=True)

            @pl.when(jj + 1 < CH // 2)
            def _():
                pltpu.make_async_copy(ra, acc.at[ua.at[1]], ssa).wait()
                _unpack_both(pk_v, j0 + 2, ua)
                pltpu.async_copy(table_hbm.at[ua.at[0]], ra, sa)
                pltpu.make_async_copy(rb, acc.at[ub.at[1]], ssb).wait()
                _unpack_both(pk_v, j0 + 3, ub)
                pltpu.async_copy(table_hbm.at[ub.at[0]], rb, sb)

            @pl.when(jj + 1 >= CH // 2)
            def _():
                pltpu.make_async_copy(ra, acc.at[ua.at[1]], ssa).wait()
                pltpu.make_async_copy(rb, acc.at[ub.at[1]], ssb).wait()

            return carry

        lax.fori_loop(0, CH // 2, body, 0)
        plsc.subcore_barrier()
        pltpu.sync_copy(acc.at[pl.ds(s, STR)], out_hbm.at[cid].at[pl.ds(s, STR)])

    return k(table, pk3, zeros_row)


def _tc_matmul(a, w):
    n, din = a.shape
    dout = w.shape[1]

    def body(a_ref, w_ref, o_ref):
        o_ref[...] = jnp.dot(a_ref[...], w_ref[...],
                             preferred_element_type=jnp.float32)

    return pl.pallas_call(
        body,
        grid=(n // R,),
        in_specs=[
            pl.BlockSpec((R, din), lambda i: (i, 0)),
            pl.BlockSpec((din, dout), lambda i: (0, 0)),
        ],
        out_specs=pl.BlockSpec((R, dout), lambda i: (i, 0)),
        out_shape=jax.ShapeDtypeStruct((n, dout), jnp.float32),
    )(a, w)


def _tc_dinv(degp):
    def body(p_ref, dinv_ref):
        deg = 1.0 + p_ref[0] + p_ref[1]
        dinv_ref[...] = jnp.broadcast_to(lax.rsqrt(deg)[:, None],
                                         (N_PADD, DEGW))

    return pl.pallas_call(
        body,
        in_specs=[pl.BlockSpec((NC, N_PADD), lambda: (0, 0))],
        out_specs=pl.BlockSpec((N_PADD, DEGW), lambda: (0, 0)),
        out_shape=jax.ShapeDtypeStruct((N_PADD, DEGW), jnp.float32),
    )(degp)


def _tc_scale1(dinv16, g1):
    def body(dinv_ref, g_ref, h_ref):
        h_ref[...] = g_ref[...] * dinv_ref[:, :1]

    return pl.pallas_call(
        body,
        grid=(N // R,),
        in_specs=[
            pl.BlockSpec((R, DEGW), lambda i: (i, 0)),
            pl.BlockSpec((R, D), lambda i: (i, 0)),
        ],
        out_specs=pl.BlockSpec((R, D), lambda i: (i, 0)),
        out_shape=jax.ShapeDtypeStruct((N, D), jnp.float32),
    )(dinv16, g1)


def _tc_layer_mid(segp, ht, dinv16, b, w):
    def body(p_ref, ht_ref, dinv_ref, b_ref, w_ref, h_ref, ht2_ref):
        dinv = dinv_ref[:, :1]
        s = p_ref[0] + p_ref[1] + ht_ref[...]
        h = jnp.maximum(s * dinv + b_ref[...], 0.0)
        h_ref[...] = h
        ht2_ref[...] = jnp.dot(h, w_ref[...],
                               preferred_element_type=jnp.float32) * dinv

    return pl.pallas_call(
        body,
        grid=(N // R,),
        in_specs=[
            pl.BlockSpec((NC, R, D), lambda i: (0, i, 0)),
            pl.BlockSpec((R, D), lambda i: (i, 0)),
            pl.BlockSpec((R, DEGW), lambda i: (i, 0)),
            pl.BlockSpec((1, D), lambda i: (0, 0)),
            pl.BlockSpec((D, D), lambda i: (0, 0)),
        ],
        out_specs=[
            pl.BlockSpec((R, D), lambda i: (i, 0)),
            pl.BlockSpec((R, D), lambda i: (i, 0)),
        ],
        out_shape=[
            jax.ShapeDtypeStruct((N, D), jnp.float32),
            jax.ShapeDtypeStruct((N, D), jnp.float32),
        ],
    )(segp, ht, dinv16, b, w)


def _tc_layer_out(segp, ht, dinv16, b, wc, bc):
    def body(p_ref, ht_ref, dinv_ref, b_ref, wc_ref, bc_ref, h_ref, lg_ref):
        dinv = dinv_ref[:, :1]
        h = jnp.maximum((p_ref[0] + p_ref[1] + ht_ref[...]) * dinv
                        + b_ref[...], 0.0)
        h_ref[...] = h
        lg_ref[...] = jnp.dot(h, wc_ref[...],
                              preferred_element_type=jnp.float32) + bc_ref[...]

    return pl.pallas_call(
        body,
        grid=(N // R,),
        in_specs=[
            pl.BlockSpec((NC, R, D), lambda i: (0, i, 0)),
            pl.BlockSpec((R, D), lambda i: (i, 0)),
            pl.BlockSpec((R, DEGW), lambda i: (i, 0)),
            pl.BlockSpec((1, D), lambda i: (0, 0)),
            pl.BlockSpec((D, C), lambda i: (0, 0)),
            pl.BlockSpec((1, C), lambda i: (0, 0)),
        ],
        out_specs=[
            pl.BlockSpec((R, D), lambda i: (i, 0)),
            pl.BlockSpec((R, C), lambda i: (i, 0)),
        ],
        out_shape=[
            jax.ShapeDtypeStruct((N, D), jnp.float32),
            jax.ShapeDtypeStruct((N, C), jnp.float32),
        ],
    )(segp, ht, dinv16, b, wc, bc)


def kernel(x, edge_index, W1, b1, W2, b2, Wc, bc):
    pad = EPW_P - EPW
    srcw = jnp.pad(edge_index[0].reshape(NW, EPW), ((0, 0), (0, pad)))
    pad_dst = N + (jnp.arange(pad, dtype=jnp.int32) % (N_PAD - N))
    dstw = jnp.concatenate(
        [edge_index[1].reshape(NW, EPW),
         jnp.broadcast_to(pad_dst, (NW, pad))], axis=1)
    pk3 = (srcw | (dstw << SHIFT)).reshape(NW, CH, K)
    zeros_row = jnp.zeros((STR, D), jnp.float32)
    ones1 = jnp.ones((K,), jnp.float32)
    zeros1 = jnp.zeros((STRD,), jnp.float32)

    g1 = _tc_matmul(x, W1)
    degp = _sc_degree(pk3, ones1, zeros1)
    dinv16 = _tc_dinv(degp)
    h1t = _tc_scale1(dinv16, g1)
    seg1 = _sc_scatter_rows(h1t, pk3, zeros_row)
    h1, h2t = _tc_layer_mid(seg1, h1t, dinv16, b1.reshape(1, D), W2)
    seg2 = _sc_scatter_rows(h2t, pk3, zeros_row)
    h2, logits = _tc_layer_out(seg2, h2t, dinv16, b2.reshape(1, D),
                               Wc, bc.reshape(1, C))
    feat_list = jnp.concatenate([h1, h2, logits], axis=1)
    return (logits, feat_list)


# speedup vs baseline: 1.1269x; 1.1269x over previous
import functools

import jax
import jax.numpy as jnp
from jax import lax
from jax.experimental import pallas as pl
from jax.experimental.pallas import tpu as pltpu
from jax.experimental.pallas import tpu_sc as plsc

N = 10000
E = 320000
D = 128
C = 16
NC = 2
NS = 16
NW = NC * NS
EPW = E // NW
K = 128
EPW_P = 10240
CH = EPW_P // K
N_PAD = 10112
STR = N_PAD // NS
N_PADD = 12288
STRD = N_PADD // NS
DEGW = 16
SHIFT = 14
MASK = (1 << SHIFT) - 1
R = 1000


def _sc_mesh():
    return plsc.VectorSubcoreMesh(core_axis_name="c", subcore_axis_name="s")


def _unpack_dst(pk_v, j, u, row):
    for i in range(K // 16):
        v = pk_v[j, pl.ds(i * 16, 16)]
        u[row, pl.ds(i * 16, 16)] = lax.shift_right_logical(v, SHIFT)


def _unpack_both(pk_v, j, u):
    for i in range(K // 16):
        v = pk_v[j, pl.ds(i * 16, 16)]
        u[0, pl.ds(i * 16, 16)] = v & MASK
        u[1, pl.ds(i * 16, 16)] = lax.shift_right_logical(v, SHIFT)


def _sc_degree(pk3, ones_k, zeros_k):
    @functools.partial(
        pl.kernel,
        out_type=jax.ShapeDtypeStruct((NC, N_PADD), jnp.float32),
        mesh=_sc_mesh(),
        scratch_types=[
            pltpu.VMEM((CH, K), jnp.int32),
            pltpu.VMEM((2, K), jnp.int32),
            pltpu.VMEM((K,), jnp.float32),
            pltpu.VMEM_SHARED((N_PADD,), jnp.float32),
        ],
    )
    def k(pk_hbm, ones_hbm, zeros_hbm, out_hbm, pk_v, u, ones_v, acc):
        cid = lax.axis_index("c")
        sid = lax.axis_index("s")
        wid = sid * NC + cid
        s = sid * STRD
        pltpu.sync_copy(pk_hbm.at[wid], pk_v)
        pltpu.sync_copy(ones_hbm, ones_v)
        pltpu.sync_copy(zeros_hbm, acc.at[pl.ds(s, STRD)])
        plsc.subcore_barrier()

        def body(j, carry):
            _unpack_dst(pk_v, j, u, 0)
            pltpu.sync_copy(ones_v, acc.at[u.at[0]], add=True)
            return carry

        lax.fori_loop(0, CH, body, 0)
        plsc.subcore_barrier()
        pltpu.sync_copy(acc.at[pl.ds(s, STRD)], out_hbm.at[cid].at[pl.ds(s, STRD)])

    return k(pk3, ones_k, zeros_k)


def _sc_scatter_rows(table, pk3, zeros_row):
    @functools.partial(
        pl.kernel,
        out_type=jax.ShapeDtypeStruct((NC, N_PAD, D), jnp.float32),
        mesh=_sc_mesh(),
        scratch_types=[
            pltpu.VMEM((CH, K), jnp.int32),
            pltpu.VMEM((2, K), jnp.int32),
            pltpu.VMEM((2, K), jnp.int32),
            pltpu.VMEM((K, D), jnp.float32),
            pltpu.VMEM((K, D), jnp.float32),
            pltpu.VMEM_SHARED((N_PAD, D), jnp.float32),
            pltpu.SemaphoreType.DMA,
            pltpu.SemaphoreType.DMA,
        ],
    )
    def k(table_hbm, pk_hbm, z_hbm, out_hbm,
          pk_v, ua, ub, ra, rb, acc, sa, sb):
        cid = lax.axis_index("c")
        sid = lax.axis_index("s")
        wid = sid * NC + cid
        s = sid * STR
        pltpu.sync_copy(pk_hbm.at[wid], pk_v)
        pltpu.sync_copy(z_hbm, acc.at[pl.ds(s, STR)])
        plsc.subcore_barrier()

        _unpack_both(pk_v, 0, ua)
        pltpu.async_copy(table_hbm.at[ua.at[0]], ra, sa)
        _unpack_both(pk_v, 1, ub)
        pltpu.async_copy(table_hbm.at[ub.at[0]], rb, sb)

        def body(jj, carry):
            j0 = jj * 2
            pltpu.make_async_copy(table_hbm.at[ua.at[0]], ra, sa).wait()
            pltpu.sync_copy(ra, acc.at[ua.at[1]], add=True)

            @pl.when(jj + 1 < CH // 2)
            def _():
                _unpack_both(pk_v, j0 + 2, ua)
                pltpu.async_copy(table_hbm.at[ua.at[0]], ra, sa)

            pltpu.make_async_copy(table_hbm.at[ub.at[0]], rb, sb).wait()
            pltpu.sync_copy(rb, acc.at[ub.at[1]], add=True)

            @pl.when(jj + 1 < CH // 2)
            def _():
                _unpack_both(pk_v, j0 + 3, ub)
                pltpu.async_copy(table_hbm.at[ub.at[0]], rb, sb)

            return carry

        lax.fori_loop(0, CH // 2, body, 0)
        plsc.subcore_barrier()
        pltpu.sync_copy(acc.at[pl.ds(s, STR)], out_hbm.at[cid].at[pl.ds(s, STR)])

    return k(table, pk3, zeros_row)


def _tc_pack_edges(src2, dst2):
    WB = 8
    pad = EPW_P - EPW

    def body(s_ref, d_ref, o_ref):
        pk = s_ref[...] | (d_ref[...] << SHIFT)
        it = lax.broadcasted_iota(jnp.int32, (WB, pad), 1)
        pv = (N + lax.rem(it, N_PAD - N)) << SHIFT
        o_ref[...] = jnp.concatenate([pk, pv], axis=1)

    return pl.pallas_call(
        body,
        grid=(NW // WB,),
        in_specs=[
            pl.BlockSpec((WB, EPW), lambda i: (i, 0)),
            pl.BlockSpec((WB, EPW), lambda i: (i, 0)),
        ],
        out_specs=pl.BlockSpec((WB, EPW_P), lambda i: (i, 0)),
        out_shape=jax.ShapeDtypeStruct((NW, EPW_P), jnp.int32),
    )(src2, dst2)


def _tc_matmul(a, w):
    n, din = a.shape
    dout = w.shape[1]

    def body(a_ref, w_ref, o_ref):
        o_ref[...] = jnp.dot(a_ref[...], w_ref[...],
                             preferred_element_type=jnp.float32)

    return pl.pallas_call(
        body,
        grid=(n // R,),
        in_specs=[
            pl.BlockSpec((R, din), lambda i: (i, 0)),
            pl.BlockSpec((din, dout), lambda i: (0, 0)),
        ],
        out_specs=pl.BlockSpec((R, dout), lambda i: (i, 0)),
        out_shape=jax.ShapeDtypeStruct((n, dout), jnp.float32),
    )(a, w)


def _tc_dinv(degp):
    def body(p_ref, dinv_ref):
        deg = 1.0 + p_ref[0] + p_ref[1]
        dinv_ref[...] = jnp.broadcast_to(lax.rsqrt(deg)[:, None],
                                         (N_PADD, DEGW))

    return pl.pallas_call(
        body,
        in_specs=[pl.BlockSpec((NC, N_PADD), lambda: (0, 0))],
        out_specs=pl.BlockSpec((N_PADD, DEGW), lambda: (0, 0)),
        out_shape=jax.ShapeDtypeStruct((N_PADD, DEGW), jnp.float32),
    )(degp)


def _tc_scale1(dinv16, g1):
    def body(dinv_ref, g_ref, h_ref):
        h_ref[...] = g_ref[...] * dinv_ref[:, :1]

    return pl.pallas_call(
        body,
        grid=(N // R,),
        in_specs=[
            pl.BlockSpec((R, DEGW), lambda i: (i, 0)),
            pl.BlockSpec((R, D), lambda i: (i, 0)),
        ],
        out_specs=pl.BlockSpec((R, D), lambda i: (i, 0)),
        out_shape=jax.ShapeDtypeStruct((N, D), jnp.float32),
    )(dinv16, g1)


def _tc_layer_mid(segp, ht, dinv16, b, w):
    def body(p_ref, ht_ref, dinv_ref, b_ref, w_ref, h_ref, ht2_ref):
        dinv = dinv_ref[:, :1]
        s = p_ref[0] + p_ref[1] + ht_ref[...]
        h = jnp.maximum(s * dinv + b_ref[...], 0.0)
        h_ref[...] = h
        ht2_ref[...] = jnp.dot(h, w_ref[...],
                               preferred_element_type=jnp.float32) * dinv

    return pl.pallas_call(
        body,
        grid=(N // R,),
        in_specs=[
            pl.BlockSpec((NC, R, D), lambda i: (0, i, 0)),
            pl.BlockSpec((R, D), lambda i: (i, 0)),
            pl.BlockSpec((R, DEGW), lambda i: (i, 0)),
            pl.BlockSpec((1, D), lambda i: (0, 0)),
            pl.BlockSpec((D, D), lambda i: (0, 0)),
        ],
        out_specs=[
            pl.BlockSpec((R, D), lambda i: (i, 0)),
            pl.BlockSpec((R, D), lambda i: (i, 0)),
        ],
        out_shape=[
            jax.ShapeDtypeStruct((N, D), jnp.float32),
            jax.ShapeDtypeStruct((N, D), jnp.float32),
        ],
    )(segp, ht, dinv16, b, w)


def _tc_layer_out(segp, ht, dinv16, b, wc, bc, h1):
    def body(p_ref, ht_ref, dinv_ref, b_ref, wc_ref, bc_ref, h1_ref,
             lg_ref, f_ref):
        dinv = dinv_ref[:, :1]
        h = jnp.maximum((p_ref[0] + p_ref[1] + ht_ref[...]) * dinv
                        + b_ref[...], 0.0)
        lg = jnp.dot(h, wc_ref[...],
                     preferred_element_type=jnp.float32) + bc_ref[...]
        lg_ref[...] = lg
        f_ref[...] = jnp.concatenate([h1_ref[...], h, lg], axis=1)

    return pl.pallas_call(
        body,
        grid=(N // R,),
        in_specs=[
            pl.BlockSpec((NC, R, D), lambda i: (0, i, 0)),
            pl.BlockSpec((R, D), lambda i: (i, 0)),
            pl.BlockSpec((R, DEGW), lambda i: (i, 0)),
            pl.BlockSpec((1, D), lambda i: (0, 0)),
            pl.BlockSpec((D, C), lambda i: (0, 0)),
            pl.BlockSpec((1, C), lambda i: (0, 0)),
            pl.BlockSpec((R, D), lambda i: (i, 0)),
        ],
        out_specs=[
            pl.BlockSpec((R, C), lambda i: (i, 0)),
            pl.BlockSpec((R, 2 * D + C), lambda i: (i, 0)),
        ],
        out_shape=[
            jax.ShapeDtypeStruct((N, C), jnp.float32),
            jax.ShapeDtypeStruct((N, 2 * D + C), jnp.float32),
        ],
    )(segp, ht, dinv16, b, wc, bc, h1)


def kernel(x, edge_index, W1, b1, W2, b2, Wc, bc):
    pk3 = _tc_pack_edges(edge_index[0].reshape(NW, EPW),
                         edge_index[1].reshape(NW, EPW)).reshape(NW, CH, K)
    zeros_row = jnp.zeros((STR, D), jnp.float32)
    ones1 = jnp.ones((K,), jnp.float32)
    zeros1 = jnp.zeros((STRD,), jnp.float32)

    g1 = _tc_matmul(x, W1)
    degp = _sc_degree(pk3, ones1, zeros1)
    dinv16 = _tc_dinv(degp)
    h1t = _tc_scale1(dinv16, g1)
    seg1 = _sc_scatter_rows(h1t, pk3, zeros_row)
    h1, h2t = _tc_layer_mid(seg1, h1t, dinv16, b1.reshape(1, D), W2)
    seg2 = _sc_scatter_rows(h2t, pk3, zeros_row)
    logits, feat_list = _tc_layer_out(seg2, h2t, dinv16, b2.reshape(1, D),
                                      Wc, bc.reshape(1, C), h1)
    return (logits, feat_list)


# speedup vs baseline: 1.1574x; 1.0270x over previous
import functools

import jax
import jax.numpy as jnp
from jax import lax
from jax.experimental import pallas as pl
from jax.experimental.pallas import tpu as pltpu
from jax.experimental.pallas import tpu_sc as plsc

N = 10000
E = 320000
D = 128
C = 16
NC = 2
NS = 16
NW = NC * NS
EPW = E // NW
K = 128
EPW_P = 10240
CH = EPW_P // K
KS = 64
NSLOT = 4
CHS = EPW_P // KS
N_PAD = 10112
STR = N_PAD // NS
N_PADD = 12288
STRD = N_PADD // NS
DEGW = 16
SHIFT = 14
MASK = (1 << SHIFT) - 1
R = 1000


def _sc_mesh():
    return plsc.VectorSubcoreMesh(core_axis_name="c", subcore_axis_name="s")


def _unpack_dst(pk_v, base, ud, n):
    for i in range(n // 16):
        v = pk_v[pl.ds(base + i * 16, 16)]
        ud[pl.ds(i * 16, 16)] = lax.shift_right_logical(v, SHIFT)


def _unpack_both(pk_v, base, us, ud, n):
    for i in range(n // 16):
        v = pk_v[pl.ds(base + i * 16, 16)]
        us[pl.ds(i * 16, 16)] = v & MASK
        ud[pl.ds(i * 16, 16)] = lax.shift_right_logical(v, SHIFT)


def _sc_degree(pk3, ones_k, zeros_k):
    @functools.partial(
        pl.kernel,
        out_type=jax.ShapeDtypeStruct((NC, N_PADD), jnp.float32),
        mesh=_sc_mesh(),
        scratch_types=[
            pltpu.VMEM((EPW_P,), jnp.int32),
            pltpu.VMEM((K,), jnp.int32),
            pltpu.VMEM((K,), jnp.float32),
            pltpu.VMEM_SHARED((N_PADD,), jnp.float32),
        ],
    )
    def k(pk_hbm, ones_hbm, zeros_hbm, out_hbm, pk_v, ud, ones_v, acc):
        cid = lax.axis_index("c")
        sid = lax.axis_index("s")
        wid = sid * NC + cid
        s = sid * STRD
        pltpu.sync_copy(pk_hbm.at[wid], pk_v)
        pltpu.sync_copy(ones_hbm, ones_v)
        pltpu.sync_copy(zeros_hbm, acc.at[pl.ds(s, STRD)])
        plsc.subcore_barrier()

        def body(j, carry):
            _unpack_dst(pk_v, j * K, ud, K)
            pltpu.sync_copy(ones_v, acc.at[ud], add=True)
            return carry

        lax.fori_loop(0, CH, body, 0)
        plsc.subcore_barrier()
        pltpu.sync_copy(acc.at[pl.ds(s, STRD)], out_hbm.at[cid].at[pl.ds(s, STRD)])

    return k(pk3, ones_k, zeros_k)


def _sc_scatter_rows(table, pk3, zeros_row):
    @functools.partial(
        pl.kernel,
        out_type=jax.ShapeDtypeStruct((NC, N_PAD, D), jnp.float32),
        mesh=_sc_mesh(),
        scratch_types=(
            [pltpu.VMEM((EPW_P,), jnp.int32)]
            + [pltpu.VMEM((KS,), jnp.int32)] * (2 * NSLOT)
            + [pltpu.VMEM((KS, D), jnp.float32)] * NSLOT
            + [pltpu.VMEM_SHARED((N_PAD, D), jnp.float32)]
            + [pltpu.SemaphoreType.DMA] * NSLOT
        ),
    )
    def k(table_hbm, pk_hbm, z_hbm, out_hbm, pk_v,
          us0, us1, us2, us3, ud0, ud1, ud2, ud3,
          r0, r1, r2, r3, acc, s0, s1, s2, s3):
        us = [us0, us1, us2, us3]
        ud = [ud0, ud1, ud2, ud3]
        rr = [r0, r1, r2, r3]
        ss = [s0, s1, s2, s3]
        cid = lax.axis_index("c")
        sid = lax.axis_index("s")
        wid = sid * NC + cid
        s = sid * STR
        pltpu.sync_copy(pk_hbm.at[wid], pk_v)
        pltpu.sync_copy(z_hbm, acc.at[pl.ds(s, STR)])
        plsc.subcore_barrier()

        for t in range(NSLOT):
            _unpack_both(pk_v, t * KS, us[t], ud[t], KS)
            pltpu.async_copy(table_hbm.at[us[t]], rr[t], ss[t])

        def body(q, carry):
            j0 = q * NSLOT
            for t in range(NSLOT):
                pltpu.make_async_copy(table_hbm.at[us[t]], rr[t], ss[t]).wait()
                pltpu.sync_copy(rr[t], acc.at[ud[t]], add=True)

                @pl.when(q + 1 < CHS // NSLOT)
                def _(t=t):
                    base = (j0 + NSLOT + t) * KS
                    _unpack_both(pk_v, base, us[t], ud[t], KS)
                    pltpu.async_copy(table_hbm.at[us[t]], rr[t], ss[t])

            return carry

        lax.fori_loop(0, CHS // NSLOT, body, 0)
        plsc.subcore_barrier()
        pltpu.sync_copy(acc.at[pl.ds(s, STR)], out_hbm.at[cid].at[pl.ds(s, STR)])

    return k(table, pk3, zeros_row)


def _tc_pack_edges(src2, dst2):
    WB = 8
    pad = EPW_P - EPW

    def body(s_ref, d_ref, o_ref):
        pk = s_ref[...] | (d_ref[...] << SHIFT)
        it = lax.broadcasted_iota(jnp.int32, (WB, pad), 1)
        pv = (N + lax.rem(it, N_PAD - N)) << SHIFT
        o_ref[...] = jnp.concatenate([pk, pv], axis=1)

    return pl.pallas_call(
        body,
        grid=(NW // WB,),
        in_specs=[
            pl.BlockSpec((WB, EPW), lambda i: (i, 0)),
            pl.BlockSpec((WB, EPW), lambda i: (i, 0)),
        ],
        out_specs=pl.BlockSpec((WB, EPW_P), lambda i: (i, 0)),
        out_shape=jax.ShapeDtypeStruct((NW, EPW_P), jnp.int32),
    )(src2, dst2)


def _tc_matmul(a, w):
    n, din = a.shape
    dout = w.shape[1]

    def body(a_ref, w_ref, o_ref):
        o_ref[...] = jnp.dot(a_ref[...], w_ref[...],
                             preferred_element_type=jnp.float32)

    return pl.pallas_call(
        body,
        grid=(n // R,),
        in_specs=[
            pl.BlockSpec((R, din), lambda i: (i, 0)),
            pl.BlockSpec((din, dout), lambda i: (0, 0)),
        ],
        out_specs=pl.BlockSpec((R, dout), lambda i: (i, 0)),
        out_shape=jax.ShapeDtypeStruct((n, dout), jnp.float32),
    )(a, w)


def _tc_dinv(degp):
    def body(p_ref, dinv_ref):
        deg = 1.0 + p_ref[0] + p_ref[1]
        dinv_ref[...] = jnp.broadcast_to(lax.rsqrt(deg)[:, None],
                                         (N_PADD, DEGW))

    return pl.pallas_call(
        body,
        in_specs=[pl.BlockSpec((NC, N_PADD), lambda: (0, 0))],
        out_specs=pl.BlockSpec((N_PADD, DEGW), lambda: (0, 0)),
        out_shape=jax.ShapeDtypeStruct((N_PADD, DEGW), jnp.float32),
    )(degp)


def _tc_scale1(dinv16, g1):
    def body(dinv_ref, g_ref, h_ref):
        h_ref[...] = g_ref[...] * dinv_ref[:, :1]

    return pl.pallas_call(
        body,
        grid=(N // R,),
        in_specs=[
            pl.BlockSpec((R, DEGW), lambda i: (i, 0)),
            pl.BlockSpec((R, D), lambda i: (i, 0)),
        ],
        out_specs=pl.BlockSpec((R, D), lambda i: (i, 0)),
        out_shape=jax.ShapeDtypeStruct((N, D), jnp.float32),
    )(dinv16, g1)


def _tc_layer_mid(segp, ht, dinv16, b, w):
    def body(p_ref, ht_ref, dinv_ref, b_ref, w_ref, h_ref, ht2_ref):
        dinv = dinv_ref[:, :1]
        s = p_ref[0] + p_ref[1] + ht_ref[...]
        h = jnp.maximum(s * dinv + b_ref[...], 0.0)
        h_ref[...] = h
        ht2_ref[...] = jnp.dot(h, w_ref[...],
                               preferred_element_type=jnp.float32) * dinv

    return pl.pallas_call(
        body,
        grid=(N // R,),
        in_specs=[
            pl.BlockSpec((NC, R, D), lambda i: (0, i, 0)),
            pl.BlockSpec((R, D), lambda i: (i, 0)),
            pl.BlockSpec((R, DEGW), lambda i: (i, 0)),
            pl.BlockSpec((1, D), lambda i: (0, 0)),
            pl.BlockSpec((D, D), lambda i: (0, 0)),
        ],
        out_specs=[
            pl.BlockSpec((R, D), lambda i: (i, 0)),
            pl.BlockSpec((R, D), lambda i: (i, 0)),
        ],
        out_shape=[
            jax.ShapeDtypeStruct((N, D), jnp.float32),
            jax.ShapeDtypeStruct((N, D), jnp.float32),
        ],
    )(segp, ht, dinv16, b, w)


def _tc_layer_out(segp, ht, dinv16, b, wc, bc, h1):
    def body(p_ref, ht_ref, dinv_ref, b_ref, wc_ref, bc_ref, h1_ref,
             lg_ref, f_ref):
        dinv = dinv_ref[:, :1]
        h = jnp.maximum((p_ref[0] + p_ref[1] + ht_ref[...]) * dinv
                        + b_ref[...], 0.0)
        lg = jnp.dot(h, wc_ref[...],
                     preferred_element_type=jnp.float32) + bc_ref[...]
        lg_ref[...] = lg
        f_ref[...] = jnp.concatenate([h1_ref[...], h, lg], axis=1)

    return pl.pallas_call(
        body,
        grid=(N // R,),
        in_specs=[
            pl.BlockSpec((NC, R, D), lambda i: (0, i, 0)),
            pl.BlockSpec((R, D), lambda i: (i, 0)),
            pl.BlockSpec((R, DEGW), lambda i: (i, 0)),
            pl.BlockSpec((1, D), lambda i: (0, 0)),
            pl.BlockSpec((D, C), lambda i: (0, 0)),
            pl.BlockSpec((1, C), lambda i: (0, 0)),
            pl.BlockSpec((R, D), lambda i: (i, 0)),
        ],
        out_specs=[
            pl.BlockSpec((R, C), lambda i: (i, 0)),
            pl.BlockSpec((R, 2 * D + C), lambda i: (i, 0)),
        ],
        out_shape=[
            jax.ShapeDtypeStruct((N, C), jnp.float32),
            jax.ShapeDtypeStruct((N, 2 * D + C), jnp.float32),
        ],
    )(segp, ht, dinv16, b, wc, bc, h1)


def kernel(x, edge_index, W1, b1, W2, b2, Wc, bc):
    pk3 = _tc_pack_edges(edge_index[0].reshape(NW, EPW),
                         edge_index[1].reshape(NW, EPW))
    zeros_row = jnp.zeros((STR, D), jnp.float32)
    ones1 = jnp.ones((K,), jnp.float32)
    zeros1 = jnp.zeros((STRD,), jnp.float32)

    g1 = _tc_matmul(x, W1)
    degp = _sc_degree(pk3, ones1, zeros1)
    dinv16 = _tc_dinv(degp)
    h1t = _tc_scale1(dinv16, g1)
    seg1 = _sc_scatter_rows(h1t, pk3, zeros_row)
    h1, h2t = _tc_layer_mid(seg1, h1t, dinv16, b1.reshape(1, D), W2)
    seg2 = _sc_scatter_rows(h2t, pk3, zeros_row)
    logits, feat_list = _tc_layer_out(seg2, h2t, dinv16, b2.reshape(1, D),
                                      Wc, bc.reshape(1, C), h1)
    return (logits, feat_list)


# speedup vs baseline: 1.1586x; 1.0010x over previous
import functools

import jax
import jax.numpy as jnp
from jax import lax
from jax.experimental import pallas as pl
from jax.experimental.pallas import tpu as pltpu
from jax.experimental.pallas import tpu_sc as plsc

N = 10000
E = 320000
D = 128
C = 16
NC = 2
NS = 16
NW = NC * NS
EPW = E // NW
K = 128
EPW_P = 10240
CH = EPW_P // K
KS = 64
NSLOT = 4
CHS = EPW_P // KS
N_PAD = 10112
STR = N_PAD // NS
N_PADD = 12288
STRD = N_PADD // NS
DEGW = 16
SHIFT = 14
MASK = (1 << SHIFT) - 1
R = 1000


def _sc_mesh():
    return plsc.VectorSubcoreMesh(core_axis_name="c", subcore_axis_name="s")


def _unpack_dst(pk_v, base, ud, n):
    for i in range(n // 16):
        v = pk_v[pl.ds(base + i * 16, 16)]
        ud[pl.ds(i * 16, 16)] = lax.shift_right_logical(v, SHIFT)


def _unpack_both(pk_v, base, us, ud, n):
    for i in range(n // 16):
        v = pk_v[pl.ds(base + i * 16, 16)]
        us[pl.ds(i * 16, 16)] = v & MASK
        ud[pl.ds(i * 16, 16)] = lax.shift_right_logical(v, SHIFT)


def _sc_degree(pk3, ones_k, zeros_k):
    @functools.partial(
        pl.kernel,
        out_type=jax.ShapeDtypeStruct((NC, N_PADD), jnp.float32),
        mesh=_sc_mesh(),
        scratch_types=[
            pltpu.VMEM((EPW_P,), jnp.int32),
            pltpu.VMEM((K,), jnp.int32),
            pltpu.VMEM((K,), jnp.float32),
            pltpu.VMEM_SHARED((N_PADD,), jnp.float32),
        ],
    )
    def k(pk_hbm, ones_hbm, zeros_hbm, out_hbm, pk_v, ud, ones_v, acc):
        cid = lax.axis_index("c")
        sid = lax.axis_index("s")
        wid = sid * NC + cid
        s = sid * STRD
        pltpu.sync_copy(pk_hbm.at[wid], pk_v)
        pltpu.sync_copy(ones_hbm, ones_v)
        pltpu.sync_copy(zeros_hbm, acc.at[pl.ds(s, STRD)])
        plsc.subcore_barrier()

        def body(j, carry):
            _unpack_dst(pk_v, j * K, ud, K)
            pltpu.sync_copy(ones_v, acc.at[ud], add=True)
            return carry

        lax.fori_loop(0, CH, body, 0)
        plsc.subcore_barrier()
        pltpu.sync_copy(acc.at[pl.ds(s, STRD)], out_hbm.at[cid].at[pl.ds(s, STRD)])

    return k(pk3, ones_k, zeros_k)


def _sc_scatter_rows(table, pk3, zeros_row):
    @functools.partial(
        pl.kernel,
        out_type=jax.ShapeDtypeStruct((NC, N_PAD, D), jnp.float32),
        mesh=_sc_mesh(),
        scratch_types=(
            [pltpu.VMEM((EPW_P,), jnp.int32)]
            + [pltpu.VMEM((KS,), jnp.int32)] * (2 * NSLOT)
            + [pltpu.VMEM((KS, D), jnp.float32)] * NSLOT
            + [pltpu.VMEM_SHARED((N_PAD, D), jnp.float32)]
            + [pltpu.SemaphoreType.DMA] * NSLOT
        ),
    )
    def k(table_hbm, pk_hbm, z_hbm, out_hbm, pk_v,
          us0, us1, us2, us3, ud0, ud1, ud2, ud3,
          r0, r1, r2, r3, acc, s0, s1, s2, s3):
        us = [us0, us1, us2, us3]
        ud = [ud0, ud1, ud2, ud3]
        rr = [r0, r1, r2, r3]
        ss = [s0, s1, s2, s3]
        cid = lax.axis_index("c")
        sid = lax.axis_index("s")
        wid = sid * NC + cid
        s = sid * STR
        pltpu.sync_copy(pk_hbm.at[wid], pk_v)
        pltpu.sync_copy(z_hbm, acc.at[pl.ds(s, STR)])
        plsc.subcore_barrier()

        for t in range(NSLOT):
            _unpack_both(pk_v, t * KS, us[t], ud[t], KS)
            pltpu.async_copy(table_hbm.at[us[t]], rr[t], ss[t])

        def body(q, carry):
            j0 = q * NSLOT
            for t in range(NSLOT):
                pltpu.make_async_copy(table_hbm.at[us[t]], rr[t], ss[t]).wait()
                pltpu.sync_copy(rr[t], acc.at[ud[t]], add=True)

                @pl.when(q + 1 < CHS // NSLOT)
                def _(t=t):
                    base = (j0 + NSLOT + t) * KS
                    _unpack_both(pk_v, base, us[t], ud[t], KS)
                    pltpu.async_copy(table_hbm.at[us[t]], rr[t], ss[t])

            return carry

        lax.fori_loop(0, CHS // NSLOT, body, 0)
        plsc.subcore_barrier()
        pltpu.sync_copy(acc.at[pl.ds(s, STR)], out_hbm.at[cid].at[pl.ds(s, STR)])

    return k(table, pk3, zeros_row)


def _tc_pack_edges(src2, dst2):
    WB = 8
    pad = EPW_P - EPW

    def body(s_ref, d_ref, o_ref):
        pk = s_ref[...] | (d_ref[...] << SHIFT)
        it = lax.broadcasted_iota(jnp.int32, (WB, pad), 1)
        pv = (N + lax.rem(it, N_PAD - N)) << SHIFT
        o_ref[...] = jnp.concatenate([pk, pv], axis=1)

    return pl.pallas_call(
        body,
        grid=(NW // WB,),
        in_specs=[
            pl.BlockSpec((WB, EPW), lambda i: (i, 0)),
            pl.BlockSpec((WB, EPW), lambda i: (i, 0)),
        ],
        out_specs=pl.BlockSpec((WB, EPW_P), lambda i: (i, 0)),
        out_shape=jax.ShapeDtypeStruct((NW, EPW_P), jnp.int32),
    )(src2, dst2)


def _tc_dinv(degp):
    def body(p_ref, dinv_ref):
        deg = 1.0 + p_ref[0] + p_ref[1]
        dinv_ref[...] = jnp.broadcast_to(lax.rsqrt(deg)[:, None],
                                         (N_PADD, DEGW))

    return pl.pallas_call(
        body,
        in_specs=[pl.BlockSpec((NC, N_PADD), lambda: (0, 0))],
        out_specs=pl.BlockSpec((N_PADD, DEGW), lambda: (0, 0)),
        out_shape=jax.ShapeDtypeStruct((N_PADD, DEGW), jnp.float32),
    )(degp)


def _tc_scale1(dinv16, x, w):
    def body(dinv_ref, x_ref, w_ref, h_ref):
        g = jnp.dot(x_ref[...], w_ref[...], preferred_element_type=jnp.float32)
        h_ref[...] = g * dinv_ref[:, :1]

    return pl.pallas_call(
        body,
        grid=(N // R,),
        in_specs=[
            pl.BlockSpec((R, DEGW), lambda i: (i, 0)),
            pl.BlockSpec((R, D), lambda i: (i, 0)),
            pl.BlockSpec((D, D), lambda i: (0, 0)),
        ],
        out_specs=pl.BlockSpec((R, D), lambda i: (i, 0)),
        out_shape=jax.ShapeDtypeStruct((N, D), jnp.float32),
    )(dinv16, x, w)


def _tc_layer_mid(segp, ht, dinv16, b, w):
    def body(p_ref, ht_ref, dinv_ref, b_ref, w_ref, h_ref, ht2_ref):
        dinv = dinv_ref[:, :1]
        s = p_ref[0] + p_ref[1] + ht_ref[...]
        h = jnp.maximum(s * dinv + b_ref[...], 0.0)
        h_ref[...] = h
        ht2_ref[...] = jnp.dot(h, w_ref[...],
                               preferred_element_type=jnp.float32) * dinv

    return pl.pallas_call(
        body,
        grid=(N // R,),
        in_specs=[
            pl.BlockSpec((NC, R, D), lambda i: (0, i, 0)),
            pl.BlockSpec((R, D), lambda i: (i, 0)),
            pl.BlockSpec((R, DEGW), lambda i: (i, 0)),
            pl.BlockSpec((1, D), lambda i: (0, 0)),
            pl.BlockSpec((D, D), lambda i: (0, 0)),
        ],
        out_specs=[
            pl.BlockSpec((R, D), lambda i: (i, 0)),
            pl.BlockSpec((R, D), lambda i: (i, 0)),
        ],
        out_shape=[
            jax.ShapeDtypeStruct((N, D), jnp.float32),
            jax.ShapeDtypeStruct((N, D), jnp.float32),
        ],
    )(segp, ht, dinv16, b, w)


def _tc_layer_out(segp, ht, dinv16, b, wc, bc, h1):
    def body(p_ref, ht_ref, dinv_ref, b_ref, wc_ref, bc_ref, h1_ref,
             lg_ref, f_ref):
        dinv = dinv_ref[:, :1]
        h = jnp.maximum((p_ref[0] + p_ref[1] + ht_ref[...]) * dinv
                        + b_ref[...], 0.0)
        lg = jnp.dot(h, wc_ref[...],
                     preferred_element_type=jnp.float32) + bc_ref[...]
        lg_ref[...] = lg
        f_ref[...] = jnp.concatenate([h1_ref[...], h, lg], axis=1)

    return pl.pallas_call(
        body,
        grid=(N // R,),
        in_specs=[
            pl.BlockSpec((NC, R, D), lambda i: (0, i, 0)),
            pl.BlockSpec((R, D), lambda i: (i, 0)),
            pl.BlockSpec((R, DEGW), lambda i: (i, 0)),
            pl.BlockSpec((1, D), lambda i: (0, 0)),
            pl.BlockSpec((D, C), lambda i: (0, 0)),
            pl.BlockSpec((1, C), lambda i: (0, 0)),
            pl.BlockSpec((R, D), lambda i: (i, 0)),
        ],
        out_specs=[
            pl.BlockSpec((R, C), lambda i: (i, 0)),
            pl.BlockSpec((R, 2 * D + C), lambda i: (i, 0)),
        ],
        out_shape=[
            jax.ShapeDtypeStruct((N, C), jnp.float32),
            jax.ShapeDtypeStruct((N, 2 * D + C), jnp.float32),
        ],
    )(segp, ht, dinv16, b, wc, bc, h1)


def kernel(x, edge_index, W1, b1, W2, b2, Wc, bc):
    pk3 = _tc_pack_edges(edge_index[0].reshape(NW, EPW),
                         edge_index[1].reshape(NW, EPW))
    zeros_row = jnp.zeros((STR, D), jnp.float32)
    ones1 = jnp.ones((K,), jnp.float32)
    zeros1 = jnp.zeros((STRD,), jnp.float32)

    degp = _sc_degree(pk3, ones1, zeros1)
    dinv16 = _tc_dinv(degp)
    h1t = _tc_scale1(dinv16, x, W1)
    seg1 = _sc_scatter_rows(h1t, pk3, zeros_row)
    h1, h2t = _tc_layer_mid(seg1, h1t, dinv16, b1.reshape(1, D), W2)
    seg2 = _sc_scatter_rows(h2t, pk3, zeros_row)
    logits, feat_list = _tc_layer_out(seg2, h2t, dinv16, b2.reshape(1, D),
                                      Wc, bc.reshape(1, C), h1)
    return (logits, feat_list)
